# Initial kernel scaffold; baseline (speedup 1.0000x reference)
import jax
import jax.numpy as jnp
from jax import lax
from jax.experimental import pallas as pl
from jax.experimental.pallas import tpu as pltpu
from jax.experimental.pallas import tpu_sc as plsc

N = 10000
E = 320000
H = 8
C = 16
F = 128
ED = 16
OUT = 16

NC = 2
NS = 16
NW = NC * NS
K = 128
EP = 327680
EW = EP // NW
CH = EW // K
NP = N + 16
ZR = NP // NS

_mesh = plsc.VectorSubcoreMesh(core_axis_name="c", subcore_axis_name="s",
                               num_cores=NC, num_subcores=NS)

_f32 = jnp.float32


def _lrelu(v):
    return jnp.where(v >= 0.0, v, 0.2 * v)


def _splat(v, h):
    dn = lax.GatherDimensionNumbers(offset_dims=(), collapsed_slice_dims=(0,),
                                    start_index_map=(0,))
    idx = jnp.full((16, 1), h, jnp.int32)
    return lax.gather(v, idx, dn, (1,),
                      mode=lax.GatherScatterMode.PROMISE_IN_BOUNDS)


def _make_pass1(with_cnt, lcol):
    out_type = [jax.ShapeDtypeStruct((EP, 16), _f32),
                jax.ShapeDtypeStruct((NC, NP, 16), _f32),
                jax.ShapeDtypeStruct((NC, NP, 16), _f32)]
    scratch = [pltpu.VMEM((K,), jnp.int32),
               pltpu.VMEM((K,), jnp.int32),
               pltpu.VMEM((K, 16), _f32),
               pltpu.VMEM((K, 16), _f32),
               pltpu.VMEM((K, 16), _f32),
               pltpu.VMEM((K, 16), _f32),
               pltpu.VMEM((ZR, 16), _f32),
               pltpu.VMEM_SHARED((NP, 16), _f32),
               pltpu.VMEM_SHARED((NP, 16), _f32)]
    if with_cnt:
        out_type.append(jax.ShapeDtypeStruct((NC, NP, 16), _f32))
        scratch.append(pltpu.VMEM((K, 16), _f32))
        scratch.append(pltpu.VMEM_SHARED((NP, 16), _f32))

    def body(src, dst, a16, d16p, ae, exo, deno, eso, *rest):
        if with_cnt:
            cnto, srcv, dstv, sav, sdv, aev, exv, zb, den_sp, es_sp, onesv, cnt_sp = rest
        else:
            srcv, dstv, sav, sdv, aev, exv, zb, den_sp, es_sp = rest
        c = lax.axis_index("c")
        s = lax.axis_index("s")
        wid = c * NS + s

        def zb_body(i, carry):
            zb[i] = jnp.zeros((16,), _f32)
            return carry
        lax.fori_loop(0, ZR, zb_body, 0)
        if with_cnt:
            def ones_body(i, carry):
                onesv[i] = jnp.ones((16,), _f32)
                return carry
            lax.fori_loop(0, K, ones_body, 0)
        base_r = s * ZR
        pltpu.sync_copy(zb, den_sp.at[pl.ds(base_r, ZR), :])
        pltpu.sync_copy(zb, es_sp.at[pl.ds(base_r, ZR), :])
        if with_cnt:
            pltpu.sync_copy(zb, cnt_sp.at[pl.ds(base_r, ZR), :])
        plsc.subcore_barrier()

        def chunk(j, carry):
            base = wid * EW + j * K
            pltpu.sync_copy(src.at[pl.ds(base, K)], srcv)
            pltpu.sync_copy(dst.at[pl.ds(base, K)], dstv)
            pltpu.sync_copy(ae.at[pl.ds(base, K), pl.ds(lcol, 16)], aev)
            pltpu.sync_copy(a16.at[srcv], sav)
            pltpu.sync_copy(d16p.at[dstv], sdv)

            def ebody(e, carry2):
                v = sav[e] + sdv[e] + aev[e]
                exv[e] = jnp.exp(_lrelu(v))
                return carry2
            lax.fori_loop(0, K, ebody, 0)
            pltpu.sync_copy(exv, exo.at[pl.ds(base, K), :])
            pltpu.sync_copy(exv, den_sp.at[dstv], add=True)
            pltpu.sync_copy(aev, es_sp.at[dstv], add=True)
            if with_cnt:
                pltpu.sync_copy(onesv, cnt_sp.at[dstv], add=True)
            return carry
        lax.fori_loop(0, CH, chunk, 0)
        plsc.subcore_barrier()

        @pl.when(s == 0)
        def _():
            pltpu.sync_copy(den_sp, deno.at[c])
            pltpu.sync_copy(es_sp, eso.at[c])
            if with_cnt:
                pltpu.sync_copy(cnt_sp, cnto.at[c])

    return pl.kernel(body, out_type=tuple(out_type), mesh=_mesh,
                     scratch_types=tuple(scratch))


_pass1_l1 = _make_pass1(True, 0)
_pass1_l2 = _make_pass1(False, 16)


def _pass2_body(src, dst, ex, xl, outo,
                srcv, dstv, exv, Xv, zb, out_sp):
    c = lax.axis_index("c")
    s = lax.axis_index("s")
    wid = c * NS + s

    def zb_body(i, carry):
        for k in range(8):
            zb[i, pl.ds(k * 16, 16)] = jnp.zeros((16,), _f32)
        return carry
    lax.fori_loop(0, ZR // 2, zb_body, 0)
    base_r = s * ZR
    pltpu.sync_copy(zb, out_sp.at[pl.ds(base_r, ZR // 2), :])
    pltpu.sync_copy(zb, out_sp.at[pl.ds(base_r + ZR // 2, ZR // 2), :])
    plsc.subcore_barrier()

    def chunk(j, carry):
        base = wid * EW + j * K
        pltpu.sync_copy(src.at[pl.ds(base, K)], srcv)
        pltpu.sync_copy(dst.at[pl.ds(base, K)], dstv)
        pltpu.sync_copy(ex.at[pl.ds(base, K), :], exv)
        pltpu.sync_copy(xl.at[srcv], Xv)

        def ebody(e, carry2):
            ev = exv[e]
            for h in range(H):
                sl = pl.ds(h * 16, 16)
                Xv[e, sl] = Xv[e, sl] * _splat(ev, h)
            return carry2
        lax.fori_loop(0, K, ebody, 0)
        pltpu.sync_copy(Xv, out_sp.at[dstv], add=True)
        return carry
    lax.fori_loop(0, CH, chunk, 0)
    plsc.subcore_barrier()

    @pl.when(s == 0)
    def _():
        pltpu.sync_copy(out_sp, outo.at[c])


_pass2 = pl.kernel(
    _pass2_body,
    out_type=jax.ShapeDtypeStruct((NC, NP, F), _f32),
    mesh=_mesh,
    scratch_types=(pltpu.VMEM((K,), jnp.int32),
                   pltpu.VMEM((K,), jnp.int32),
                   pltpu.VMEM((K, 16), _f32),
                   pltpu.VMEM((K, F), _f32),
                   pltpu.VMEM((ZR // 2, F), _f32),
                   pltpu.VMEM_SHARED((NP, F), _f32)))


_B = 2000
_EB = 8192


def _premix1_body(x_ref, wcat_ref, xl_ref, a_ref, d_ref):
    p = jnp.dot(x_ref[...], wcat_ref[...], preferred_element_type=_f32)
    xl_ref[...] = p[:, :F]
    a_ref[...] = p[:, F:F + 16]
    d_ref[...] = p[:, F + 16:F + 32]


def _edgepre_body(ea_ref, aecat_ref, out_ref):
    out_ref[...] = jnp.dot(ea_ref[...], aecat_ref[...],
                           preferred_element_type=_f32)


def _combine_body(outp, den, es, cnt, a16, d16, xl, rep, bvec, hraw, stats):
    step = pl.program_id(0)
    den8 = den[0, :, :8] + den[1, :, :8]
    es8 = es[0, :, :8] + es[1, :, :8]
    cnt8 = cnt[0, :, :8] + cnt[1, :, :8]
    al = a16[:, :8] + d16[:, :8] + es8 / jnp.maximum(cnt8, 1.0)
    exl = jnp.exp(_lrelu(al))
    dtot = den8 + exl
    o = outp[0] + outp[1]
    hr = (o + xl[...] * jnp.dot(exl, rep[...], preferred_element_type=_f32)) \
        / jnp.dot(dtot, rep[...], preferred_element_type=_f32) + bvec[...]
    hraw[...] = hr
    st = jnp.concatenate([jnp.sum(hr, axis=0, keepdims=True),
                          jnp.sum(hr * hr, axis=0, keepdims=True)], axis=0)

    @pl.when(step == 0)
    def _():
        stats[...] = st

    @pl.when(step > 0)
    def _():
        stats[...] += st


def _bn_elu(h_blk, stats):
    m = stats[0:1, :] / N
    v = stats[1:2, :] / N - m * m
    xin = (h_blk - m) / jnp.sqrt(v + 1e-5)
    return jnp.where(xin > 0, xin, jnp.exp(xin) - 1.0)


def _premix2_body(hraw, stats, wcat, xl_ref, a_ref, d_ref):
    xin = _bn_elu(hraw[...], stats[...])
    p = jnp.dot(xin, wcat[...], preferred_element_type=_f32)
    xl_ref[...] = p[:, :F]
    a_ref[...] = p[:, F:F + 16]
    d_ref[...] = p[:, F + 16:F + 32]


def _head_body(hraw, stats, wl, blv, out_ref):
    xin = _bn_elu(hraw[...], stats[...])
    out_ref[...] = jnp.dot(xin, wl[...], preferred_element_type=_f32) + blv[...]


def _node_spec(width):
    return pl.BlockSpec((_B, width), lambda i: (i, 0))


def _full_spec(shape):
    return pl.BlockSpec(shape, lambda i: tuple(0 for _ in shape))


_premix = pl.pallas_call(
    _premix1_body,
    grid=(N // _B,),
    in_specs=[_node_spec(F), _full_spec((F, F + 32))],
    out_specs=[_node_spec(F), _node_spec(16), _node_spec(16)],
    out_shape=[jax.ShapeDtypeStruct((N, F), _f32),
               jax.ShapeDtypeStruct((N, 16), _f32),
               jax.ShapeDtypeStruct((N, 16), _f32)],
)

_edgepre = pl.pallas_call(
    _edgepre_body,
    grid=(EP // _EB,),
    in_specs=[pl.BlockSpec((_EB, ED), lambda i: (i, 0)), _full_spec((ED, 32))],
    out_specs=pl.BlockSpec((_EB, 32), lambda i: (i, 0)),
    out_shape=jax.ShapeDtypeStruct((EP, 32), _f32),
)

_combine = pl.pallas_call(
    _combine_body,
    grid=(N // _B,),
    in_specs=[pl.BlockSpec((NC, _B, F), lambda i: (0, i, 0)),
              pl.BlockSpec((NC, _B, 16), lambda i: (0, i, 0)),
              pl.BlockSpec((NC, _B, 16), lambda i: (0, i, 0)),
              pl.BlockSpec((NC, _B, 16), lambda i: (0, i, 0)),
              _node_spec(16), _node_spec(16), _node_spec(F),
              _full_spec((H, F)), _full_spec((1, F))],
    out_specs=[_node_spec(F), _full_spec((2, F))],
    out_shape=[jax.ShapeDtypeStruct((N, F), _f32),
               jax.ShapeDtypeStruct((2, F), _f32)],
)

_premix_next = pl.pallas_call(
    _premix2_body,
    grid=(N // _B,),
    in_specs=[_node_spec(F), _full_spec((2, F)), _full_spec((F, F + 32))],
    out_specs=[_node_spec(F), _node_spec(16), _node_spec(16)],
    out_shape=[jax.ShapeDtypeStruct((N, F), _f32),
               jax.ShapeDtypeStruct((N, 16), _f32),
               jax.ShapeDtypeStruct((N, 16), _f32)],
)

_head = pl.pallas_call(
    _head_body,
    grid=(N // _B,),
    in_specs=[_node_spec(F), _full_spec((2, F)), _full_spec((F, OUT)),
              _full_spec((1, OUT))],
    out_specs=_node_spec(OUT),
    out_shape=jax.ShapeDtypeStruct((N, OUT), _f32),
)


def _fold(W, att):
    a = (W.reshape(W.shape[0], H, C) * att[None]).sum(-1)
    return jnp.concatenate([a, a], axis=1)


def kernel(x, edge_index, edge_attr, W1, as1, ad1, We1, ae1, b1,
           W2, as2, ad2, We2, ae2, b2, Wl, bl):
    src = edge_index[0]
    dst = edge_index[1]
    pad = EP - E
    src_p = jnp.concatenate([src, jnp.zeros((pad,), jnp.int32)])
    dst_p = jnp.concatenate([dst, jnp.full((pad,), N, jnp.int32)])
    ea_p = jnp.concatenate([edge_attr, jnp.zeros((pad, ED), _f32)], axis=0)

    aecat = jnp.concatenate([_fold(We1, ae1), _fold(We2, ae2)], axis=1)
    wcat1 = jnp.concatenate([W1, _fold(W1, as1), _fold(W1, ad1)], axis=1)
    wcat2 = jnp.concatenate([W2, _fold(W2, as2), _fold(W2, ad2)], axis=1)
    rep = jnp.zeros((H, F), _f32)
    rep = rep.at[jnp.repeat(jnp.arange(H), C), jnp.arange(F)].set(1.0)

    AE = _edgepre(ea_p, aecat)
    xl1, a1, d1 = _premix(x, wcat1)
    d1p = jnp.concatenate([d1, jnp.zeros((NP - N, 16), _f32)], axis=0)

    ex1, den1, es1, cntp = _pass1_l1(src_p, dst_p, a1, d1p, AE)
    out1 = _pass2(src_p, dst_p, ex1, xl1)
    h1, st1 = _combine(out1, den1, es1, cntp, a1, d1, xl1, rep,
                       b1.reshape(1, F))

    xl2, a2, d2 = _premix_next(h1, st1, wcat2)
    d2p = jnp.concatenate([d2, jnp.zeros((NP - N, 16), _f32)], axis=0)

    ex2, den2, es2 = _pass1_l2(src_p, dst_p, a2, d2p, AE)
    out2 = _pass2(src_p, dst_p, ex2, xl2)
    h2, st2 = _combine(out2, den2, es2, cntp, a2, d2, xl2, rep,
                       b2.reshape(1, F))

    return _head(h2, st2, Wl, bl.reshape(1, OUT))


# speedup vs baseline: 33.1739x; 33.1739x over previous
import jax
import jax.numpy as jnp
from jax import lax
from jax.experimental import pallas as pl
from jax.experimental.pallas import tpu as pltpu
from jax.experimental.pallas import tpu_sc as plsc

N = 10000
E = 320000
H = 8
C = 16
F = 128
ED = 16
OUT = 16

NC = 2
NS = 16
NW = NC * NS
K = 128
EP = 327680
EW = EP // NW
CH = EW // K
NP = N + 16
ZR = NP // NS
ZB = 64

_mesh = plsc.VectorSubcoreMesh(core_axis_name="c", subcore_axis_name="s",
                               num_cores=NC, num_subcores=NS)

_sc_params = pltpu.CompilerParams(use_tc_tiling_on_sc=False)

_f32 = jnp.float32


def _lrelu(v):
    return jnp.where(v >= 0.0, v, 0.2 * v)


def _zero_stripe(zb, sp_ref, base_r):
    for k in range(ZR // ZB):
        pltpu.sync_copy(zb, sp_ref.at[pl.ds(base_r + k * ZB, ZB), :])
    rem = ZR - (ZR // ZB) * ZB
    pltpu.sync_copy(zb.at[pl.ds(0, rem)],
                    sp_ref.at[pl.ds(base_r + (ZR // ZB) * ZB, rem), :])


def _splat(v, h):
    dn = lax.GatherDimensionNumbers(offset_dims=(), collapsed_slice_dims=(0,),
                                    start_index_map=(0,))
    idx = jnp.full((16, 1), h, jnp.int32)
    return lax.gather(v, idx, dn, (1,),
                      mode=lax.GatherScatterMode.PROMISE_IN_BOUNDS)


def _make_pass1(with_cnt, lcol):
    out_type = [jax.ShapeDtypeStruct((EP, 16), _f32),
                jax.ShapeDtypeStruct((NC, NP, 16), _f32),
                jax.ShapeDtypeStruct((NC, NP, 16), _f32)]
    scratch = [pltpu.VMEM((K,), jnp.int32),
               pltpu.VMEM((K,), jnp.int32),
               pltpu.VMEM((K, 16), _f32),
               pltpu.VMEM((K, 16), _f32),
               pltpu.VMEM((K, 16), _f32),
               pltpu.VMEM((K, 16), _f32),
               pltpu.VMEM((ZB, 16), _f32),
               pltpu.VMEM_SHARED((NP, 16), _f32),
               pltpu.VMEM_SHARED((NP, 16), _f32)]
    if with_cnt:
        out_type.append(jax.ShapeDtypeStruct((NC, NP, 16), _f32))
        scratch.append(pltpu.VMEM((K, 16), _f32))
        scratch.append(pltpu.VMEM_SHARED((NP, 16), _f32))

    def body(src, dst, a16, d16p, ae, exo, deno, eso, *rest):
        if with_cnt:
            cnto, srcv, dstv, sav, sdv, aev, exv, zb, den_sp, es_sp, onesv, cnt_sp = rest
        else:
            srcv, dstv, sav, sdv, aev, exv, zb, den_sp, es_sp = rest
        c = lax.axis_index("c")
        s = lax.axis_index("s")
        wid = c * NS + s

        def zb_body(i, carry):
            zb[i] = jnp.zeros((16,), _f32)
            return carry
        lax.fori_loop(0, ZB, zb_body, 0)
        if with_cnt:
            def ones_body(i, carry):
                onesv[i] = jnp.ones((16,), _f32)
                return carry
            lax.fori_loop(0, K, ones_body, 0)
        base_r = s * ZR
        _zero_stripe(zb, den_sp, base_r)
        _zero_stripe(zb, es_sp, base_r)
        if with_cnt:
            _zero_stripe(zb, cnt_sp, base_r)
        plsc.subcore_barrier()

        def chunk(j, carry):
            base = wid * EW + j * K
            pltpu.sync_copy(src.at[pl.ds(base, K)], srcv)
            pltpu.sync_copy(dst.at[pl.ds(base, K)], dstv)
            pltpu.sync_copy(ae.at[pl.ds(base, K), pl.ds(lcol, 16)], aev)
            pltpu.sync_copy(a16.at[srcv], sav)
            pltpu.sync_copy(d16p.at[dstv], sdv)

            def ebody(e, carry2):
                v = sav[e] + sdv[e] + aev[e]
                exv[e] = jnp.exp(_lrelu(v))
                return carry2
            lax.fori_loop(0, K, ebody, 0)
            pltpu.sync_copy(exv, exo.at[pl.ds(base, K), :])
            pltpu.sync_copy(exv, den_sp.at[dstv], add=True)
            pltpu.sync_copy(aev, es_sp.at[dstv], add=True)
            if with_cnt:
                pltpu.sync_copy(onesv, cnt_sp.at[dstv], add=True)
            return carry
        lax.fori_loop(0, CH, chunk, 0)
        plsc.subcore_barrier()

        @pl.when(s == 0)
        def _():
            pltpu.sync_copy(den_sp, deno.at[c])
            pltpu.sync_copy(es_sp, eso.at[c])
            if with_cnt:
                pltpu.sync_copy(cnt_sp, cnto.at[c])

    return pl.kernel(body, out_type=tuple(out_type), mesh=_mesh,
                     scratch_types=tuple(scratch), compiler_params=_sc_params)


_pass1_l1 = _make_pass1(True, 0)
_pass1_l2 = _make_pass1(False, 16)


def _pass2_body(src, dst, ex, xl, outo,
                srcv, dstv, exv, Xv, zb, out_sp):
    c = lax.axis_index("c")
    s = lax.axis_index("s")
    wid = c * NS + s

    def zb_body(i, carry):
        for k in range(8):
            zb[i, pl.ds(k * 16, 16)] = jnp.zeros((16,), _f32)
        return carry
    lax.fori_loop(0, ZB, zb_body, 0)
    base_r = s * ZR
    _zero_stripe(zb, out_sp, base_r)
    plsc.subcore_barrier()

    def chunk(j, carry):
        base = wid * EW + j * K
        pltpu.sync_copy(src.at[pl.ds(base, K)], srcv)
        pltpu.sync_copy(dst.at[pl.ds(base, K)], dstv)
        pltpu.sync_copy(ex.at[pl.ds(base, K), :], exv)
        pltpu.sync_copy(xl.at[srcv], Xv)

        def ebody(e, carry2):
            ev = exv[e]
            for h in range(H):
                sl = pl.ds(h * 16, 16)
                Xv[e, sl] = Xv[e, sl] * _splat(ev, h)
            return carry2
        lax.fori_loop(0, K, ebody, 0)
        pltpu.sync_copy(Xv, out_sp.at[dstv], add=True)
        return carry
    lax.fori_loop(0, CH, chunk, 0)
    plsc.subcore_barrier()

    @pl.when(s == 0)
    def _():
        pltpu.sync_copy(out_sp, outo.at[c])


_pass2 = pl.kernel(
    _pass2_body,
    out_type=jax.ShapeDtypeStruct((NC, NP, F), _f32),
    mesh=_mesh,
    scratch_types=(pltpu.VMEM((K,), jnp.int32),
                   pltpu.VMEM((K,), jnp.int32),
                   pltpu.VMEM((K, 16), _f32),
                   pltpu.VMEM((K, F), _f32),
                   pltpu.VMEM((ZB, F), _f32),
                   pltpu.VMEM_SHARED((NP, F), _f32)),
    compiler_params=_sc_params)


_B = 2000
_EB = 8192


def _premix1_body(x_ref, wcat_ref, xl_ref, a_ref, d_ref):
    p = jnp.dot(x_ref[...], wcat_ref[...], preferred_element_type=_f32)
    xl_ref[...] = p[:, :F]
    a_ref[...] = p[:, F:F + 16]
    d_ref[...] = p[:, F + 16:F + 32]


def _edgepre_body(ea_ref, aecat_ref, out_ref):
    out_ref[...] = jnp.dot(ea_ref[...], aecat_ref[...],
                           preferred_element_type=_f32)


def _combine_body(outp, den, es, cnt, a16, d16, xl, rep, bvec, hraw, stats):
    step = pl.program_id(0)
    den8 = den[0, :, :8] + den[1, :, :8]
    es8 = es[0, :, :8] + es[1, :, :8]
    cnt8 = cnt[0, :, :8] + cnt[1, :, :8]
    al = a16[:, :8] + d16[:, :8] + es8 / jnp.maximum(cnt8, 1.0)
    exl = jnp.exp(_lrelu(al))
    dtot = den8 + exl
    o = outp[0] + outp[1]
    hr = (o + xl[...] * jnp.dot(exl, rep[...], preferred_element_type=_f32)) \
        / jnp.dot(dtot, rep[...], preferred_element_type=_f32) + bvec[...]
    hraw[...] = hr
    st = jnp.concatenate([jnp.sum(hr, axis=0, keepdims=True),
                          jnp.sum(hr * hr, axis=0, keepdims=True)], axis=0)

    @pl.when(step == 0)
    def _():
        stats[...] = st

    @pl.when(step > 0)
    def _():
        stats[...] += st


def _bn_elu(h_blk, stats):
    m = stats[0:1, :] / N
    v = stats[1:2, :] / N - m * m
    xin = (h_blk - m) / jnp.sqrt(v + 1e-5)
    return jnp.where(xin > 0, xin, jnp.exp(xin) - 1.0)


def _premix2_body(hraw, stats, wcat, xl_ref, a_ref, d_ref):
    xin = _bn_elu(hraw[...], stats[...])
    p = jnp.dot(xin, wcat[...], preferred_element_type=_f32)
    xl_ref[...] = p[:, :F]
    a_ref[...] = p[:, F:F + 16]
    d_ref[...] = p[:, F + 16:F + 32]


def _head_body(hraw, stats, wl, blv, out_ref):
    xin = _bn_elu(hraw[...], stats[...])
    out_ref[...] = jnp.dot(xin, wl[...], preferred_element_type=_f32) + blv[...]


def _node_spec(width):
    return pl.BlockSpec((_B, width), lambda i: (i, 0))


def _full_spec(shape):
    return pl.BlockSpec(shape, lambda i: tuple(0 for _ in shape))


_premix = pl.pallas_call(
    _premix1_body,
    grid=(N // _B,),
    in_specs=[_node_spec(F), _full_spec((F, F + 32))],
    out_specs=[_node_spec(F), _node_spec(16), _node_spec(16)],
    out_shape=[jax.ShapeDtypeStruct((N, F), _f32),
               jax.ShapeDtypeStruct((N, 16), _f32),
               jax.ShapeDtypeStruct((N, 16), _f32)],
)

_edgepre = pl.pallas_call(
    _edgepre_body,
    grid=(EP // _EB,),
    in_specs=[pl.BlockSpec((_EB, ED), lambda i: (i, 0)), _full_spec((ED, 32))],
    out_specs=pl.BlockSpec((_EB, 32), lambda i: (i, 0)),
    out_shape=jax.ShapeDtypeStruct((EP, 32), _f32),
)

_combine = pl.pallas_call(
    _combine_body,
    grid=(N // _B,),
    in_specs=[pl.BlockSpec((NC, _B, F), lambda i: (0, i, 0)),
              pl.BlockSpec((NC, _B, 16), lambda i: (0, i, 0)),
              pl.BlockSpec((NC, _B, 16), lambda i: (0, i, 0)),
              pl.BlockSpec((NC, _B, 16), lambda i: (0, i, 0)),
              _node_spec(16), _node_spec(16), _node_spec(F),
              _full_spec((H, F)), _full_spec((1, F))],
    out_specs=[_node_spec(F), _full_spec((2, F))],
    out_shape=[jax.ShapeDtypeStruct((N, F), _f32),
               jax.ShapeDtypeStruct((2, F), _f32)],
)

_premix_next = pl.pallas_call(
    _premix2_body,
    grid=(N // _B,),
    in_specs=[_node_spec(F), _full_spec((2, F)), _full_spec((F, F + 32))],
    out_specs=[_node_spec(F), _node_spec(16), _node_spec(16)],
    out_shape=[jax.ShapeDtypeStruct((N, F), _f32),
               jax.ShapeDtypeStruct((N, 16), _f32),
               jax.ShapeDtypeStruct((N, 16), _f32)],
)

_head = pl.pallas_call(
    _head_body,
    grid=(N // _B,),
    in_specs=[_node_spec(F), _full_spec((2, F)), _full_spec((F, OUT)),
              _full_spec((1, OUT))],
    out_specs=_node_spec(OUT),
    out_shape=jax.ShapeDtypeStruct((N, OUT), _f32),
)


def _fold(W, att):
    a = (W.reshape(W.shape[0], H, C) * att[None]).sum(-1)
    return jnp.concatenate([a, a], axis=1)


def kernel(x, edge_index, edge_attr, W1, as1, ad1, We1, ae1, b1,
           W2, as2, ad2, We2, ae2, b2, Wl, bl):
    src = edge_index[0]
    dst = edge_index[1]
    pad = EP - E
    src_p = jnp.concatenate([src, jnp.zeros((pad,), jnp.int32)])
    dst_p = jnp.concatenate([dst, jnp.full((pad,), N, jnp.int32)])
    ea_p = jnp.concatenate([edge_attr, jnp.zeros((pad, ED), _f32)], axis=0)

    aecat = jnp.concatenate([_fold(We1, ae1), _fold(We2, ae2)], axis=1)
    wcat1 = jnp.concatenate([W1, _fold(W1, as1), _fold(W1, ad1)], axis=1)
    wcat2 = jnp.concatenate([W2, _fold(W2, as2), _fold(W2, ad2)], axis=1)
    rep = jnp.zeros((H, F), _f32)
    rep = rep.at[jnp.repeat(jnp.arange(H), C), jnp.arange(F)].set(1.0)

    AE = _edgepre(ea_p, aecat)
    xl1, a1, d1 = _premix(x, wcat1)
    d1p = jnp.concatenate([d1, jnp.zeros((NP - N, 16), _f32)], axis=0)

    ex1, den1, es1, cntp = _pass1_l1(src_p, dst_p, a1, d1p, AE)
    out1 = _pass2(src_p, dst_p, ex1, xl1)
    h1, st1 = _combine(out1, den1, es1, cntp, a1, d1, xl1, rep,
                       b1.reshape(1, F))

    xl2, a2, d2 = _premix_next(h1, st1, wcat2)
    d2p = jnp.concatenate([d2, jnp.zeros((NP - N, 16), _f32)], axis=0)

    ex2, den2, es2 = _pass1_l2(src_p, dst_p, a2, d2p, AE)
    out2 = _pass2(src_p, dst_p, ex2, xl2)
    h2, st2 = _combine(out2, den2, es2, cntp, a2, d2, xl2, rep,
                       b2.reshape(1, F))

    return _head(h2, st2, Wl, bl.reshape(1, OUT))


# speedup vs baseline: 42.2403x; 1.2733x over previous
import jax
import jax.numpy as jnp
from jax import lax
from jax.experimental import pallas as pl
from jax.experimental.pallas import tpu as pltpu
from jax.experimental.pallas import tpu_sc as plsc

N = 10000
E = 320000
H = 8
C = 16
F = 128
ED = 16
OUT = 16

NC = 2
NS = 16
NW = NC * NS
K = 128
EP = 327680
EW = EP // NW
CH = EW // K
NP = N + 16
ZR = NP // NS
ZB = 64

_mesh = plsc.VectorSubcoreMesh(core_axis_name="c", subcore_axis_name="s",
                               num_cores=NC, num_subcores=NS)

_sc_params = pltpu.CompilerParams(use_tc_tiling_on_sc=False)

_f32 = jnp.float32


def _lrelu(v):
    return jnp.where(v >= 0.0, v, 0.2 * v)


def _zero_stripe(zb, sp_ref, base_r):
    for k in range(ZR // ZB):
        pltpu.sync_copy(zb, sp_ref.at[pl.ds(base_r + k * ZB, ZB), :])
    rem = ZR - (ZR // ZB) * ZB
    pltpu.sync_copy(zb.at[pl.ds(0, rem)],
                    sp_ref.at[pl.ds(base_r + (ZR // ZB) * ZB, rem), :])


def _splat(v, h):
    dn = lax.GatherDimensionNumbers(offset_dims=(), collapsed_slice_dims=(0,),
                                    start_index_map=(0,))
    idx = jnp.full((16, 1), h, jnp.int32)
    return lax.gather(v, idx, dn, (1,),
                      mode=lax.GatherScatterMode.PROMISE_IN_BOUNDS)


def _make_pass1(with_cnt, lcol):
    out_type = [jax.ShapeDtypeStruct((EP, 16), _f32),
                jax.ShapeDtypeStruct((NC, NP, 16), _f32),
                jax.ShapeDtypeStruct((NC, NP, 16), _f32)]
    scratch = [pltpu.VMEM((2, K), jnp.int32),
               pltpu.VMEM((2, K), jnp.int32),
               pltpu.VMEM((2, K, 16), _f32),
               pltpu.VMEM((2, K, 16), _f32),
               pltpu.VMEM((2, K, 16), _f32),
               pltpu.VMEM((2, K, 16), _f32),
               pltpu.VMEM((ZB, 16), _f32),
               pltpu.SemaphoreType.DMA,
               pltpu.SemaphoreType.DMA,
               pltpu.SemaphoreType.DMA,
               pltpu.SemaphoreType.DMA,
               pltpu.VMEM_SHARED((NP, 16), _f32),
               pltpu.VMEM_SHARED((NP, 16), _f32)]
    if with_cnt:
        out_type.append(jax.ShapeDtypeStruct((NC, NP, 16), _f32))
        scratch.append(pltpu.VMEM((K, 16), _f32))
        scratch.append(pltpu.VMEM_SHARED((NP, 16), _f32))

    def body(src, dst, a16, d16p, ae, exo, deno, eso, *rest):
        if with_cnt:
            (cnto, srcv, dstv, sav, sdv, aev, exv, zb, sa0, sa1, sd0, sd1,
             den_sp, es_sp, onesv, cnt_sp) = rest
        else:
            (srcv, dstv, sav, sdv, aev, exv, zb, sa0, sa1, sd0, sd1,
             den_sp, es_sp) = rest
        semA = (sa0, sa1)
        semD = (sd0, sd1)
        c = lax.axis_index("c")
        s = lax.axis_index("s")
        wid = c * NS + s

        def zb_body(i, carry):
            zb[i] = jnp.zeros((16,), _f32)
            return carry
        lax.fori_loop(0, ZB, zb_body, 0)
        if with_cnt:
            def ones_body(i, carry):
                onesv[i] = jnp.ones((16,), _f32)
                return carry
            lax.fori_loop(0, K, ones_body, 0)
        base_r = s * ZR
        _zero_stripe(zb, den_sp, base_r)
        _zero_stripe(zb, es_sp, base_r)
        if with_cnt:
            _zero_stripe(zb, cnt_sp, base_r)
        plsc.subcore_barrier()

        def linear(j, b):
            base = wid * EW + j * K
            pltpu.sync_copy(src.at[pl.ds(base, K)], srcv.at[b])
            pltpu.sync_copy(dst.at[pl.ds(base, K)], dstv.at[b])
            pltpu.sync_copy(ae.at[pl.ds(base, K), pl.ds(lcol, 16)], aev.at[b])

        def gather_issue(b):
            pltpu.async_copy(a16.at[srcv.at[b]], sav.at[b], semA[b])
            pltpu.async_copy(d16p.at[dstv.at[b]], sdv.at[b], semD[b])

        def gather_wait(b):
            pltpu.make_async_copy(a16.at[srcv.at[b]], sav.at[b], semA[b]).wait()
            pltpu.make_async_copy(d16p.at[dstv.at[b]], sdv.at[b], semD[b]).wait()

        def compute(b):
            def ebody(e, carry2):
                v = sav[b, e] + sdv[b, e] + aev[b, e]
                exv[b, e] = jnp.exp(_lrelu(v))
                return carry2
            lax.fori_loop(0, K, ebody, 0, unroll=2)

        def scatter_sync(j, b):
            base = wid * EW + j * K
            pltpu.sync_copy(exv.at[b], exo.at[pl.ds(base, K), :])
            pltpu.sync_copy(exv.at[b], den_sp.at[dstv.at[b]], add=True)
            pltpu.sync_copy(aev.at[b], es_sp.at[dstv.at[b]], add=True)
            if with_cnt:
                pltpu.sync_copy(onesv, cnt_sp.at[dstv.at[b]], add=True)

        linear(0, 0)
        gather_issue(0)
        linear(1, 1)
        gather_issue(1)

        def superstep(i, carry):
            j0 = 2 * i
            j1 = j0 + 1
            gather_wait(0)
            compute(0)
            scatter_sync(j0, 0)

            @pl.when(i < CH // 2 - 1)
            def _():
                linear(j0 + 2, 0)
                gather_issue(0)
            gather_wait(1)
            compute(1)
            scatter_sync(j1, 1)

            @pl.when(i < CH // 2 - 1)
            def _():
                linear(j1 + 2, 1)
                gather_issue(1)
            return carry
        lax.fori_loop(0, CH // 2, superstep, 0)
        plsc.subcore_barrier()

        @pl.when(s == 0)
        def _():
            pltpu.sync_copy(den_sp, deno.at[c])
            pltpu.sync_copy(es_sp, eso.at[c])
            if with_cnt:
                pltpu.sync_copy(cnt_sp, cnto.at[c])

    return pl.kernel(body, out_type=tuple(out_type), mesh=_mesh,
                     scratch_types=tuple(scratch), compiler_params=_sc_params)


_pass1_l1 = _make_pass1(True, 0)
_pass1_l2 = _make_pass1(False, 16)


def _pass2_body(src, dst, ex, xl, outo,
                srcv, dstv, exv, Xv, zb, sg0, sg1, out_sp):
    semG = (sg0, sg1)
    c = lax.axis_index("c")
    s = lax.axis_index("s")
    wid = c * NS + s

    def zb_body(i, carry):
        for k in range(8):
            zb[i, pl.ds(k * 16, 16)] = jnp.zeros((16,), _f32)
        return carry
    lax.fori_loop(0, ZB, zb_body, 0)
    base_r = s * ZR
    _zero_stripe(zb, out_sp, base_r)
    plsc.subcore_barrier()

    def linear(j, b):
        base = wid * EW + j * K
        pltpu.sync_copy(src.at[pl.ds(base, K)], srcv.at[b])
        pltpu.sync_copy(dst.at[pl.ds(base, K)], dstv.at[b])
        pltpu.sync_copy(ex.at[pl.ds(base, K), :], exv.at[b])

    def gather_issue(b):
        pltpu.async_copy(xl.at[srcv.at[b]], Xv.at[b], semG[b])

    def gather_wait(b):
        pltpu.make_async_copy(xl.at[srcv.at[b]], Xv.at[b], semG[b]).wait()

    def compute(b):
        def ebody(e, carry2):
            ev = exv[b, e]
            for h in range(H):
                sl = pl.ds(h * 16, 16)
                Xv[b, e, sl] = Xv[b, e, sl] * _splat(ev, h)
            return carry2
        lax.fori_loop(0, K, ebody, 0, unroll=2)

    def scatter_sync(b):
        pltpu.sync_copy(Xv.at[b], out_sp.at[dstv.at[b]], add=True)

    linear(0, 0)
    gather_issue(0)
    linear(1, 1)
    gather_issue(1)

    def superstep(i, carry):
        j0 = 2 * i
        j1 = j0 + 1
        gather_wait(0)
        compute(0)
        scatter_sync(0)

        @pl.when(i < CH // 2 - 1)
        def _():
            linear(j0 + 2, 0)
            gather_issue(0)
        gather_wait(1)
        compute(1)
        scatter_sync(1)

        @pl.when(i < CH // 2 - 1)
        def _():
            linear(j1 + 2, 1)
            gather_issue(1)
        return carry
    lax.fori_loop(0, CH // 2, superstep, 0)
    plsc.subcore_barrier()

    @pl.when(s == 0)
    def _():
        pltpu.sync_copy(out_sp, outo.at[c])


_pass2 = pl.kernel(
    _pass2_body,
    out_type=jax.ShapeDtypeStruct((NC, NP, F), _f32),
    mesh=_mesh,
    scratch_types=(pltpu.VMEM((2, K), jnp.int32),
                   pltpu.VMEM((2, K), jnp.int32),
                   pltpu.VMEM((2, K, 16), _f32),
                   pltpu.VMEM((2, K, F), _f32),
                   pltpu.VMEM((ZB, F), _f32),
                   pltpu.SemaphoreType.DMA,
                   pltpu.SemaphoreType.DMA,
                   pltpu.VMEM_SHARED((NP, F), _f32)),
    compiler_params=_sc_params)


_B = 2000
_EB = 8192


def _premix1_body(x_ref, wcat_ref, xl_ref, a_ref, d_ref):
    p = jnp.dot(x_ref[...], wcat_ref[...], preferred_element_type=_f32)
    xl_ref[...] = p[:, :F]
    a_ref[...] = p[:, F:F + 16]
    d_ref[...] = p[:, F + 16:F + 32]


def _edgepre_body(ea_ref, aecat_ref, out_ref):
    out_ref[...] = jnp.dot(ea_ref[...], aecat_ref[...],
                           preferred_element_type=_f32)


def _combine_body(outp, den, es, cnt, a16, d16, xl, rep, bvec, hraw, stats):
    step = pl.program_id(0)
    den8 = den[0, :, :8] + den[1, :, :8]
    es8 = es[0, :, :8] + es[1, :, :8]
    cnt8 = cnt[0, :, :8] + cnt[1, :, :8]
    al = a16[:, :8] + d16[:, :8] + es8 / jnp.maximum(cnt8, 1.0)
    exl = jnp.exp(_lrelu(al))
    dtot = den8 + exl
    o = outp[0] + outp[1]
    hr = (o + xl[...] * jnp.dot(exl, rep[...], preferred_element_type=_f32)) \
        / jnp.dot(dtot, rep[...], preferred_element_type=_f32) + bvec[...]
    hraw[...] = hr
    st = jnp.concatenate([jnp.sum(hr, axis=0, keepdims=True),
                          jnp.sum(hr * hr, axis=0, keepdims=True)], axis=0)

    @pl.when(step == 0)
    def _():
        stats[...] = st

    @pl.when(step > 0)
    def _():
        stats[...] += st


def _bn_elu(h_blk, stats):
    m = stats[0:1, :] / N
    v = stats[1:2, :] / N - m * m
    xin = (h_blk - m) / jnp.sqrt(v + 1e-5)
    return jnp.where(xin > 0, xin, jnp.exp(xin) - 1.0)


def _premix2_body(hraw, stats, wcat, xl_ref, a_ref, d_ref):
    xin = _bn_elu(hraw[...], stats[...])
    p = jnp.dot(xin, wcat[...], preferred_element_type=_f32)
    xl_ref[...] = p[:, :F]
    a_ref[...] = p[:, F:F + 16]
    d_ref[...] = p[:, F + 16:F + 32]


def _head_body(hraw, stats, wl, blv, out_ref):
    xin = _bn_elu(hraw[...], stats[...])
    out_ref[...] = jnp.dot(xin, wl[...], preferred_element_type=_f32) + blv[...]


def _node_spec(width):
    return pl.BlockSpec((_B, width), lambda i: (i, 0))


def _full_spec(shape):
    return pl.BlockSpec(shape, lambda i: tuple(0 for _ in shape))


_premix = pl.pallas_call(
    _premix1_body,
    grid=(N // _B,),
    in_specs=[_node_spec(F), _full_spec((F, F + 32))],
    out_specs=[_node_spec(F), _node_spec(16), _node_spec(16)],
    out_shape=[jax.ShapeDtypeStruct((N, F), _f32),
               jax.ShapeDtypeStruct((N, 16), _f32),
               jax.ShapeDtypeStruct((N, 16), _f32)],
)

_edgepre = pl.pallas_call(
    _edgepre_body,
    grid=(EP // _EB,),
    in_specs=[pl.BlockSpec((_EB, ED), lambda i: (i, 0)), _full_spec((ED, 32))],
    out_specs=pl.BlockSpec((_EB, 32), lambda i: (i, 0)),
    out_shape=jax.ShapeDtypeStruct((EP, 32), _f32),
)

_combine = pl.pallas_call(
    _combine_body,
    grid=(N // _B,),
    in_specs=[pl.BlockSpec((NC, _B, F), lambda i: (0, i, 0)),
              pl.BlockSpec((NC, _B, 16), lambda i: (0, i, 0)),
              pl.BlockSpec((NC, _B, 16), lambda i: (0, i, 0)),
              pl.BlockSpec((NC, _B, 16), lambda i: (0, i, 0)),
              _node_spec(16), _node_spec(16), _node_spec(F),
              _full_spec((H, F)), _full_spec((1, F))],
    out_specs=[_node_spec(F), _full_spec((2, F))],
    out_shape=[jax.ShapeDtypeStruct((N, F), _f32),
               jax.ShapeDtypeStruct((2, F), _f32)],
)

_premix_next = pl.pallas_call(
    _premix2_body,
    grid=(N // _B,),
    in_specs=[_node_spec(F), _full_spec((2, F)), _full_spec((F, F + 32))],
    out_specs=[_node_spec(F), _node_spec(16), _node_spec(16)],
    out_shape=[jax.ShapeDtypeStruct((N, F), _f32),
               jax.ShapeDtypeStruct((N, 16), _f32),
               jax.ShapeDtypeStruct((N, 16), _f32)],
)

_head = pl.pallas_call(
    _head_body,
    grid=(N // _B,),
    in_specs=[_node_spec(F), _full_spec((2, F)), _full_spec((F, OUT)),
              _full_spec((1, OUT))],
    out_specs=_node_spec(OUT),
    out_shape=jax.ShapeDtypeStruct((N, OUT), _f32),
)


def _fold(W, att):
    a = (W.reshape(W.shape[0], H, C) * att[None]).sum(-1)
    return jnp.concatenate([a, a], axis=1)


def kernel(x, edge_index, edge_attr, W1, as1, ad1, We1, ae1, b1,
           W2, as2, ad2, We2, ae2, b2, Wl, bl):
    src = edge_index[0]
    dst = edge_index[1]
    pad = EP - E
    src_p = jnp.concatenate([src, jnp.zeros((pad,), jnp.int32)])
    dst_p = jnp.concatenate([dst, jnp.full((pad,), N, jnp.int32)])
    ea_p = jnp.concatenate([edge_attr, jnp.zeros((pad, ED), _f32)], axis=0)

    aecat = jnp.concatenate([_fold(We1, ae1), _fold(We2, ae2)], axis=1)
    wcat1 = jnp.concatenate([W1, _fold(W1, as1), _fold(W1, ad1)], axis=1)
    wcat2 = jnp.concatenate([W2, _fold(W2, as2), _fold(W2, ad2)], axis=1)
    rep = jnp.zeros((H, F), _f32)
    rep = rep.at[jnp.repeat(jnp.arange(H), C), jnp.arange(F)].set(1.0)

    AE = _edgepre(ea_p, aecat)
    xl1, a1, d1 = _premix(x, wcat1)
    d1p = jnp.concatenate([d1, jnp.zeros((NP - N, 16), _f32)], axis=0)

    ex1, den1, es1, cntp = _pass1_l1(src_p, dst_p, a1, d1p, AE)
    out1 = _pass2(src_p, dst_p, ex1, xl1)
    h1, st1 = _combine(out1, den1, es1, cntp, a1, d1, xl1, rep,
                       b1.reshape(1, F))

    xl2, a2, d2 = _premix_next(h1, st1, wcat2)
    d2p = jnp.concatenate([d2, jnp.zeros((NP - N, 16), _f32)], axis=0)

    ex2, den2, es2 = _pass1_l2(src_p, dst_p, a2, d2p, AE)
    out2 = _pass2(src_p, dst_p, ex2, xl2)
    h2, st2 = _combine(out2, den2, es2, cntp, a2, d2, xl2, rep,
                       b2.reshape(1, F))

    return _head(h2, st2, Wl, bl.reshape(1, OUT))
